# Initial kernel scaffold; baseline (speedup 1.0000x reference)
import jax
import jax.numpy as jnp
from jax.experimental import pallas as pl


def kernel(z, edge_index):
    raise NotImplementedError("write your pallas kernel here")


# speedup vs baseline: 2.6567x; 2.6567x over previous
import functools

import jax
import jax.numpy as jnp
from jax import lax
from jax.experimental import pallas as pl
from jax.experimental.pallas import tpu as pltpu
from jax.experimental.pallas import tpu_sc as plsc

_E = 160000
_D = 256
_L = 16
_C = 128
_NW = 32
_NCHUNK = _E // _C


def _sc_body(z_hbm, src_hbm, dst_hbm, out_hbm, sidx, didx, srows, drows,
             outv, tbuf, sem):
    wid = lax.axis_index("s") * 2 + lax.axis_index("c")

    @pl.loop(wid, _NCHUNK, step=_NW)
    def _chunk(cidx):
        base = cidx * _C
        pltpu.sync_copy(src_hbm.at[pl.ds(base, _C)], sidx)
        pltpu.sync_copy(dst_hbm.at[pl.ds(base, _C)], didx)
        cp_s = pltpu.async_copy(z_hbm.at[sidx], srows, sem)
        cp_d = pltpu.async_copy(z_hbm.at[didx], drows, sem)
        cp_s.wait()
        cp_d.wait()

        lane = lax.iota(jnp.int32, _L)

        @pl.loop(0, _C // _L)
        def _group(g):
            for j in range(_L):
                e = g * _L + j
                acc = srows[e, pl.ds(0, _L)] * drows[e, pl.ds(0, _L)]
                for i in range(1, _D // _L):
                    acc = acc + (srows[e, pl.ds(i * _L, _L)]
                                 * drows[e, pl.ds(i * _L, _L)])
                tbuf[pl.ds(j * _L, _L)] = acc
            score = plsc.load_gather(tbuf, [lane * _L])
            for i in range(1, _L):
                score = score + plsc.load_gather(tbuf, [lane * _L + i])
            outv[pl.ds(g * _L, _L)] = 1.0 / (1.0 + jnp.exp(-score))

        pltpu.sync_copy(outv, out_hbm.at[pl.ds(base, _C)])


def kernel(z, edge_index):
    src = edge_index[0].astype(jnp.int32)
    dst = edge_index[1].astype(jnp.int32)
    mesh = plsc.VectorSubcoreMesh(core_axis_name="c", subcore_axis_name="s")
    run = functools.partial(
        pl.kernel,
        mesh=mesh,
        compiler_params=pltpu.CompilerParams(needs_layout_passes=False),
        out_type=jax.ShapeDtypeStruct((_E,), jnp.float32),
        scratch_types=[
            pltpu.VMEM((_C,), jnp.int32),
            pltpu.VMEM((_C,), jnp.int32),
            pltpu.VMEM((_C, _D), jnp.float32),
            pltpu.VMEM((_C, _D), jnp.float32),
            pltpu.VMEM((_C,), jnp.float32),
            pltpu.VMEM((_L * _L,), jnp.float32),
            pltpu.SemaphoreType.DMA,
        ],
    )(_sc_body)
    return run(z, src, dst)
